# baseline (speedup 1.0000x reference)
import jax
import jax.numpy as jnp
from jax.experimental import pallas as pl

_N, _D = 16384, 32
_MARGIN = 0.1
_K = 10
_N_RAND = 131072
_CHUNK = 2048


def _knn_graph(x, k):
    n = x.shape[0]
    x2 = jnp.sum(x * x, axis=-1)
    rows = jnp.arange(n)

    def chunk_fn(args):
        q, r = args
        d2 = jnp.sum(q * q, axis=-1)[:, None] - 2.0 * (q @ x.T) + x2[None, :]
        d2 = d2.at[jnp.arange(q.shape[0]), r].set(jnp.inf)
        _, idx = jax.lax.top_k(-d2, k)
        return idx

    nchunks = n // _CHUNK
    idx = jax.lax.map(chunk_fn, (x.reshape(nchunks, _CHUNK, -1), rows.reshape(nchunks, _CHUNK)))
    idx = idx.reshape(n, k)
    src = jnp.repeat(jnp.arange(n), k)
    dst = idx.reshape(-1)
    return jnp.stack([src, dst], axis=0)


def _hinge(x, edges, y):
    diff = x[edges[0]] - x[edges[1]]
    d = jnp.sqrt(jnp.sum(diff * diff, axis=-1) + 1e-12)
    loss = jnp.where(y > 0, d, jnp.maximum(0.0, _MARGIN - d))
    return jnp.mean(loss)


def _combine_kernel(a_ref, o_ref):
    o_ref[...] = jnp.sum(a_ref[...]).reshape(1, 1)


def kernel(x, track_edges, pid):
    y_sig = jnp.ones(track_edges.shape[1], dtype=jnp.int32)
    signal_loss = _hinge(x, track_edges, y_sig)

    edges_knn = _knn_graph(x, _K)
    y_knn = jnp.where(pid[edges_knn[0]] == pid[edges_knn[1]], 1, -1)
    knn_loss = _hinge(x, edges_knn, y_knn)

    edges_rand = jax.random.randint(jax.random.key(7), (2, _N_RAND), 0, x.shape[0])
    y_rand = jnp.where(pid[edges_rand[0]] == pid[edges_rand[1]], 1, -1)
    random_loss = _hinge(x, edges_rand, y_rand)

    parts = jnp.stack([signal_loss, knn_loss, random_loss]).reshape(1, 3)
    total = pl.pallas_call(
        _combine_kernel,
        out_shape=jax.ShapeDtypeStruct((1, 1), jnp.float32),
    )(parts)
    return total[0, 0]


# speedup vs baseline: 6.2560x; 6.2560x over previous
import functools

import jax
import jax.numpy as jnp
from jax.experimental import pallas as pl
from jax.experimental.pallas import tpu as pltpu

_N, _D = 16384, 32
_MARGIN = 0.1
_K = 10
_N_RAND = 131072

_R = 256
_CT = 2048
_G = 128
_NG = _N // _G
_NT = _N // _CT
_GPT = _CT // _G
_NB = _N // _R
_BIG = 1e9
_INF = float("inf")


def _knn_block_kernel(x_ref, pidc_ref, pidr_ref, out_ref,
                      m1_ref, p1_ref, m2_ref, p2_ref):
    b = pl.program_id(0)
    q = x_ref[pl.ds(b * _R, _R), :]
    ones = jnp.ones((1, _D), jnp.float32)
    dn = (((1,), (1,)), ((), ()))
    q2t = jax.lax.dot_general(ones, q * q, dn,
                              preferred_element_type=jnp.float32)

    def tile_body(t):
        xt = x_ref[pl.ds(t * _CT, _CT), :]
        x2t = jax.lax.dot_general(xt * xt, ones, dn,
                                  preferred_element_type=jnp.float32)
        s = jax.lax.dot_general(xt, q, dn,
                                preferred_element_type=jnp.float32)
        d2 = x2t + q2t - 2.0 * s
        ci = jax.lax.broadcasted_iota(jnp.int32, (_CT, _R), 0) + t * _CT
        ri = jax.lax.broadcasted_iota(jnp.int32, (_CT, _R), 1) + b * _R
        d2 = jnp.where(ci == ri, _INF, d2)

        tt = d2.reshape(_GPT, _G, _R)
        pidt = pidc_ref[pl.ds(t * _CT, _CT), :].reshape(_GPT, _G, 1)
        m1 = jnp.min(tt, axis=1)
        e1 = tt == m1[:, None, :]
        p1 = jnp.min(jnp.where(e1, pidt, _BIG), axis=1)
        t2 = jnp.where(e1, _INF, tt)
        m2 = jnp.min(t2, axis=1)
        p2 = jnp.min(jnp.where(t2 == m2[:, None, :], pidt, _BIG), axis=1)
        m1_ref[pl.ds(t * _GPT, _GPT), :] = m1
        p1_ref[pl.ds(t * _GPT, _GPT), :] = p1
        m2_ref[pl.ds(t * _GPT, _GPT), :] = m2
        p2_ref[pl.ds(t * _GPT, _GPT), :] = p2

    for t in range(_NT):
        tile_body(t)

    pid_rows = pidr_ref[0]
    gi = jax.lax.broadcasted_iota(jnp.int32, (_NG, _R), 0)

    def extract_body(carry):
        m1, p1, m2, acc = carry
        m = jnp.min(m1, axis=0, keepdims=True)
        gsel = jnp.min(jnp.where(m1 == m, gi, jnp.int32(1 << 30)),
                       axis=0, keepdims=True)
        msk = gi == gsel
        psel = jnp.min(jnp.where(msk, p1, _BIG), axis=0, keepdims=True)
        d = jnp.sqrt(jnp.maximum(m, 0.0) + 1e-12)
        term = jnp.where(psel == pid_rows, d,
                         jnp.maximum(0.0, _MARGIN - d))
        acc = acc + term
        m1 = jnp.where(msk, m2, m1)
        p1 = jnp.where(msk, p2_ref[...], p1)
        m2 = jnp.where(msk, _INF, m2)
        return m1, p1, m2, acc

    carry = (m1_ref[...], p1_ref[...], m2_ref[...],
             jnp.zeros((1, _R), jnp.float32))
    for _ in range(_K):
        carry = extract_body(carry)
    acc = carry[3]
    out_ref[...] = acc.reshape(1, 1, _R)


def _knn_hinge_sums(x, pid_f32):
    pidc = pid_f32.reshape(_N, 1)
    pidr = pid_f32.reshape(_NB, 1, _R)
    return pl.pallas_call(
        _knn_block_kernel,
        grid=(_NB,),
        in_specs=[
            pl.BlockSpec((_N, _D), lambda b: (b - b, b - b)),
            pl.BlockSpec((_N, 1), lambda b: (b - b, b - b)),
            pl.BlockSpec((1, 1, _R), lambda b: (b, b - b, b - b)),
        ],
        out_specs=pl.BlockSpec((1, 1, _R), lambda b: (b, b - b, b - b)),
        out_shape=jax.ShapeDtypeStruct((_NB, 1, _R), jnp.float32),
        scratch_shapes=[pltpu.VMEM((_NG, _R), jnp.float32)] * 4,
        compiler_params=pltpu.CompilerParams(
            dimension_semantics=("parallel",),
        ),
    )(x, pidc, pidr)


def _combine_kernel(knn_ref, sig_ref, rnd_ref, o_ref):
    knn = jnp.sum(knn_ref[...]) / jnp.float32(_N * _K)
    sig = jnp.sum(sig_ref[...]) / jnp.float32(65536)
    rnd = jnp.sum(rnd_ref[...]) / jnp.float32(_N_RAND)
    o_ref[...] = (knn + sig + rnd).reshape(1, 1)


def _hinge_terms(x, src, dst, pid_i32=None):
    diff = x[src] - x[dst]
    d = jnp.sqrt(jnp.sum(diff * diff, axis=-1) + 1e-12)
    if pid_i32 is None:
        return d
    y = pid_i32[src] == pid_i32[dst]
    return jnp.where(y, d, jnp.maximum(0.0, _MARGIN - d))


def kernel(x, track_edges, pid):
    pid_i32 = pid.astype(jnp.int32)
    pid_f32 = pid_i32.astype(jnp.float32)
    knn_sums = _knn_hinge_sums(x, pid_f32)

    ts = track_edges[0].astype(jnp.int32)
    td = track_edges[1].astype(jnp.int32)
    sig_terms = _hinge_terms(x, ts, td)

    edges_rand = jax.random.randint(jax.random.key(7), (2, _N_RAND), 0, _N)
    rnd_terms = _hinge_terms(x, edges_rand[0].astype(jnp.int32),
                             edges_rand[1].astype(jnp.int32), pid_i32)

    total = pl.pallas_call(
        _combine_kernel,
        out_shape=jax.ShapeDtypeStruct((1, 1), jnp.float32),
    )(knn_sums, sig_terms.reshape(64, 1024), rnd_terms.reshape(128, 1024))
    return total[0, 0]


# speedup vs baseline: 22.4343x; 3.5861x over previous
import functools

import jax
import jax.numpy as jnp
from jax import lax
from jax.experimental import pallas as pl
from jax.experimental.pallas import tpu as pltpu
from jax.experimental.pallas import tpu_sc as plsc

_N, _D = 16384, 32
_MARGIN = 0.1
_K = 10
_N_RAND = 131072

_R = 256
_CT = 2048
_G = 128
_NG = _N // _G
_NT = _N // _CT
_GPT = _CT // _G
_NB = _N // _R
_BIG = 1e9
_INF = float("inf")


def _knn_block_kernel(x_ref, pidc_ref, pidr_ref, out_ref,
                      m1_ref, p1_ref, m2_ref, p2_ref):
    b = pl.program_id(0)
    q = x_ref[pl.ds(b * _R, _R), :]
    ones = jnp.ones((1, _D), jnp.float32)
    dn = (((1,), (1,)), ((), ()))
    q2t = jax.lax.dot_general(ones, q * q, dn,
                              preferred_element_type=jnp.float32)

    def tile_body(t):
        xt = x_ref[pl.ds(t * _CT, _CT), :]
        x2t = jax.lax.dot_general(xt * xt, ones, dn,
                                  preferred_element_type=jnp.float32)
        s = jax.lax.dot_general(xt, q, dn,
                                preferred_element_type=jnp.float32)
        d2 = x2t + q2t - 2.0 * s
        ci = jax.lax.broadcasted_iota(jnp.int32, (_CT, _R), 0) + t * _CT
        ri = jax.lax.broadcasted_iota(jnp.int32, (_CT, _R), 1) + b * _R
        d2 = jnp.where(ci == ri, _INF, d2)

        tt = d2.reshape(_GPT, _G, _R)
        pidt = pidc_ref[pl.ds(t * _CT, _CT), :].reshape(_GPT, _G, 1)
        m1 = jnp.min(tt, axis=1)
        e1 = tt == m1[:, None, :]
        p1 = jnp.min(jnp.where(e1, pidt, _BIG), axis=1)
        t2 = jnp.where(e1, _INF, tt)
        m2 = jnp.min(t2, axis=1)
        p2 = jnp.min(jnp.where(t2 == m2[:, None, :], pidt, _BIG), axis=1)
        m1_ref[pl.ds(t * _GPT, _GPT), :] = m1
        p1_ref[pl.ds(t * _GPT, _GPT), :] = p1
        m2_ref[pl.ds(t * _GPT, _GPT), :] = m2
        p2_ref[pl.ds(t * _GPT, _GPT), :] = p2

    for t in range(_NT):
        tile_body(t)

    pid_rows = pidr_ref[0]
    gi = jax.lax.broadcasted_iota(jnp.int32, (_NG, _R), 0)

    def extract_body(carry):
        m1, p1, m2, acc = carry
        m = jnp.min(m1, axis=0, keepdims=True)
        gsel = jnp.min(jnp.where(m1 == m, gi, jnp.int32(1 << 30)),
                       axis=0, keepdims=True)
        msk = gi == gsel
        psel = jnp.min(jnp.where(msk, p1, _BIG), axis=0, keepdims=True)
        d = jnp.sqrt(jnp.maximum(m, 0.0) + 1e-12)
        term = jnp.where(psel == pid_rows, d,
                         jnp.maximum(0.0, _MARGIN - d))
        acc = acc + term
        m1 = jnp.where(msk, m2, m1)
        p1 = jnp.where(msk, p2_ref[...], p1)
        m2 = jnp.where(msk, _INF, m2)
        return m1, p1, m2, acc

    carry = (m1_ref[...], p1_ref[...], m2_ref[...],
             jnp.zeros((1, _R), jnp.float32))
    for _ in range(_K):
        carry = extract_body(carry)
    acc = carry[3]
    out_ref[...] = acc.reshape(1, 1, _R)


def _knn_hinge_sums(x, pid_f32):
    pidc = pid_f32.reshape(_N, 1)
    pidr = pid_f32.reshape(_NB, 1, _R)
    return pl.pallas_call(
        _knn_block_kernel,
        grid=(_NB,),
        in_specs=[
            pl.BlockSpec((_N, _D), lambda b: (b - b, b - b)),
            pl.BlockSpec((_N, 1), lambda b: (b - b, b - b)),
            pl.BlockSpec((1, 1, _R), lambda b: (b, b - b, b - b)),
        ],
        out_specs=pl.BlockSpec((1, 1, _R), lambda b: (b, b - b, b - b)),
        out_shape=jax.ShapeDtypeStruct((_NB, 1, _R), jnp.float32),
        scratch_shapes=[pltpu.VMEM((_NG, _R), jnp.float32)] * 4,
        compiler_params=pltpu.CompilerParams(
            dimension_semantics=("parallel",),
        ),
    )(x, pidc, pidr)


_NSIG = 65536
_B = 128
_NW = 32
_SB = _NSIG // _NW // _B
_RB = _N_RAND // _NW // _B


def _edge_kernel(x_hbm, ss_hbm, sd_hbm, rs_hbm, rd_hbm, pid_hbm, out_hbm,
                 pid_v, idx_s, idx_d, rows_s, rows_d, acc_v, sem1, sem2):
    wid = lax.axis_index("s") * 2 + lax.axis_index("c")
    pltpu.sync_copy(pid_hbm, pid_v)
    iota16 = lax.iota(jnp.int32, 16)

    def make_batch_body(src_hbm, dst_hbm, nbatch, with_pid):
        def batch_body(i, acc):
            base = (wid * nbatch + i) * _B
            pltpu.sync_copy(src_hbm.at[pl.ds(base, _B)], idx_s)
            pltpu.sync_copy(dst_hbm.at[pl.ds(base, _B)], idx_d)
            pltpu.async_copy(x_hbm.at[idx_s], rows_s, sem1).wait()
            pltpu.async_copy(x_hbm.at[idx_d], rows_d, sem2).wait()

            def grp(j, acc2):
                eids = j * 16 + iota16
                d2 = jnp.zeros((16,), jnp.float32)
                for c in range(_D):
                    cs = jnp.full((16,), c, jnp.int32)
                    a = plsc.load_gather(rows_s, [eids, cs])
                    b = plsc.load_gather(rows_d, [eids, cs])
                    df = a - b
                    d2 = d2 + df * df
                xv = d2 + 1e-12
                bits = lax.bitcast_convert_type(xv, jnp.int32)
                y = lax.bitcast_convert_type(
                    lax.shift_right_logical(bits, jnp.int32(1))
                    + jnp.int32(0x1FBD1DF5),
                    jnp.float32)
                y = 0.5 * (y + xv / y)
                y = 0.5 * (y + xv / y)
                if with_pid:
                    si = plsc.load_gather(idx_s, [eids])
                    di = plsc.load_gather(idx_d, [eids])
                    ps = plsc.load_gather(pid_v, [si])
                    pd = plsc.load_gather(pid_v, [di])
                    term = jnp.where(ps == pd, y,
                                     jnp.maximum(0.0, _MARGIN - y))
                else:
                    term = y
                return acc2 + term

            return lax.fori_loop(jnp.int32(0), jnp.int32(_B // 16), grp, acc)
        return batch_body

    zero = jnp.zeros((16,), jnp.float32)
    acc = lax.fori_loop(jnp.int32(0), jnp.int32(_SB),
                        make_batch_body(ss_hbm, sd_hbm, _SB, False), zero)
    acc_v[...] = acc
    pltpu.sync_copy(acc_v, out_hbm.at[jnp.int32(0), wid])
    acc = lax.fori_loop(jnp.int32(0), jnp.int32(_RB),
                        make_batch_body(rs_hbm, rd_hbm, _RB, True), zero)
    acc_v[...] = acc
    pltpu.sync_copy(acc_v, out_hbm.at[jnp.int32(1), wid])


def _edge_losses(x, ss, sd, rs, rd, pid_i32):
    mesh = plsc.VectorSubcoreMesh(core_axis_name="c", subcore_axis_name="s",
                                  num_cores=2, num_subcores=16)
    return pl.kernel(
        _edge_kernel,
        out_type=jax.ShapeDtypeStruct((2, _NW, 16), jnp.float32),
        mesh=mesh,
        scratch_types=[
            pltpu.VMEM((_N,), jnp.int32),
            pltpu.VMEM((_B,), jnp.int32),
            pltpu.VMEM((_B,), jnp.int32),
            pltpu.VMEM((_B, _D), jnp.float32),
            pltpu.VMEM((_B, _D), jnp.float32),
            pltpu.VMEM((16,), jnp.float32),
            pltpu.SemaphoreType.DMA,
            pltpu.SemaphoreType.DMA,
        ],
        compiler_params=pltpu.CompilerParams(needs_layout_passes=False,
                                             use_tc_tiling_on_sc=False),
    )(x, ss, sd, rs, rd, pid_i32)


def _combine_kernel(knn_ref, sc_ref, o_ref):
    knn = jnp.sum(knn_ref[...]) / jnp.float32(_N * _K)
    sig = jnp.sum(sc_ref[0]) / jnp.float32(_NSIG)
    rnd = jnp.sum(sc_ref[1]) / jnp.float32(_N_RAND)
    o_ref[...] = (knn + sig + rnd).reshape(1, 1)


def kernel(x, track_edges, pid):
    pid_i32 = pid.astype(jnp.int32)
    pid_f32 = pid_i32.astype(jnp.float32)
    knn_sums = _knn_hinge_sums(x, pid_f32)

    ts = track_edges[0].astype(jnp.int32)
    td = track_edges[1].astype(jnp.int32)
    edges_rand = jax.random.randint(jax.random.key(7), (2, _N_RAND), 0, _N)
    sc_out = _edge_losses(x, ts, td, edges_rand[0].astype(jnp.int32),
                          edges_rand[1].astype(jnp.int32), pid_i32)

    total = pl.pallas_call(
        _combine_kernel,
        out_shape=jax.ShapeDtypeStruct((1, 1), jnp.float32),
    )(knn_sums, sc_out)
    return total[0, 0]
